# initial kernel scaffold (unmeasured)
import jax
import jax.numpy as jnp
from jax import lax
from jax.experimental import pallas as pl
from jax.experimental.pallas import tpu as pltpu

N_DEV = 8


def kernel(x, w_mat):
    m_glob, k_shard = x.shape
    k_glob, n = w_mat.shape
    m_blk = m_glob // N_DEV

    def body(x_ref, w_ref, out_ref, xb, recv_buf, wbuf, amax_buf,
             send_sems, recv_sems, amax_send_sems, amax_recv_sems, wcp_sems):
        me = lax.axis_index("i")

        xb[:, :] = x_ref[:, :].astype(jnp.bfloat16)

        sends = []
        for o in range(1, N_DEV):
            dst = lax.rem(me + o, N_DEV)
            snd = pltpu.make_async_remote_copy(
                src_ref=xb.at[pl.ds(dst * m_blk, m_blk), :],
                dst_ref=recv_buf.at[me],
                send_sem=send_sems.at[o - 1],
                recv_sem=recv_sems.at[me],
                device_id=(dst,),
                device_id_type=pl.DeviceIdType.MESH,
            )
            snd.start()
            sends.append(snd)

        recv_buf[me] = xb[pl.ds(me * m_blk, m_blk), :]

        def w_copy(origin, slot):
            cp = pltpu.make_async_copy(
                w_ref.at[pl.ds(origin * k_shard, k_shard), :],
                wbuf.at[slot],
                wcp_sems.at[slot],
            )
            cp.start()
            return cp

        cps = {0: w_copy(me, 0)}
        for o in range(N_DEV):
            src = lax.rem(me - o + N_DEV, N_DEV)
            if o + 1 < N_DEV:
                nxt = lax.rem(me - o - 1 + N_DEV, N_DEV)
                cps[o + 1] = w_copy(nxt, (o + 1) % 2)
            if o > 0:
                rcv = pltpu.make_async_remote_copy(
                    src_ref=xb.at[pl.ds(0, m_blk), :],
                    dst_ref=recv_buf.at[src],
                    send_sem=send_sems.at[0],
                    recv_sem=recv_sems.at[src],
                    device_id=(me,),
                    device_id_type=pl.DeviceIdType.MESH,
                )
                rcv.wait_recv()
            cps[o].wait()
            wb = wbuf[o % 2].astype(jnp.bfloat16)
            part = jnp.dot(recv_buf[src], wb,
                           preferred_element_type=jnp.float32)
            if o == 0:
                out_ref[:, :] = part
            else:
                out_ref[:, :] = out_ref[:, :] + part

        y = jnp.maximum(out_ref[:, :], 0.0)
        out_ref[:, :] = y
        amax_buf[pl.ds(me, 1), :] = jnp.full((1, 128), jnp.max(y), jnp.float32)

        a_sends = []
        for o in range(1, N_DEV):
            dst = lax.rem(me + o, N_DEV)
            asnd = pltpu.make_async_remote_copy(
                src_ref=amax_buf.at[pl.ds(me, 1), :],
                dst_ref=amax_buf.at[pl.ds(me, 1), :],
                send_sem=amax_send_sems.at[o - 1],
                recv_sem=amax_recv_sems.at[me],
                device_id=(dst,),
                device_id_type=pl.DeviceIdType.MESH,
            )
            asnd.start()
            a_sends.append(asnd)
        for o in range(1, N_DEV):
            src = lax.rem(me - o + N_DEV, N_DEV)
            arcv = pltpu.make_async_remote_copy(
                src_ref=amax_buf.at[pl.ds(me, 1), :],
                dst_ref=amax_buf.at[pl.ds(src, 1), :],
                send_sem=amax_send_sems.at[0],
                recv_sem=amax_recv_sems.at[src],
                device_id=(me,),
                device_id_type=pl.DeviceIdType.MESH,
            )
            arcv.wait_recv()

        g_amax = jnp.max(amax_buf[:, :])
        scale = g_amax / 448.0
        q = (out_ref[:, :] / scale).astype(jnp.float8_e4m3fn)
        out_ref[:, :] = q.astype(jnp.float32) * scale

        for s in sends:
            s.wait_send()
        for s in a_sends:
            s.wait_send()

    return pl.pallas_call(
        body,
        out_shape=jax.ShapeDtypeStruct((m_blk, n), jnp.float32),
        in_specs=[
            pl.BlockSpec(memory_space=pltpu.VMEM),
            pl.BlockSpec(memory_space=pltpu.ANY),
        ],
        out_specs=pl.BlockSpec(memory_space=pltpu.VMEM),
        scratch_shapes=[
            pltpu.VMEM((m_glob, k_shard), jnp.bfloat16),
            pltpu.VMEM((N_DEV, m_blk, k_shard), jnp.bfloat16),
            pltpu.VMEM((2, k_shard, n), jnp.float32),
            pltpu.VMEM((N_DEV, 128), jnp.float32),
            pltpu.SemaphoreType.DMA((N_DEV - 1,)),
            pltpu.SemaphoreType.DMA((N_DEV,)),
            pltpu.SemaphoreType.DMA((N_DEV - 1,)),
            pltpu.SemaphoreType.DMA((N_DEV,)),
            pltpu.SemaphoreType.DMA((2,)),
        ],
        compiler_params=pltpu.CompilerParams(collective_id=0),
    )(x, w_mat)


# baseline (device time: 78173 ns/iter reference)
import jax
import jax.numpy as jnp
from jax import lax
from jax.experimental import pallas as pl
from jax.experimental.pallas import tpu as pltpu

N_DEV = 8
N_SPLIT = 2


def kernel(x, w_mat):
    m_glob, k_shard = x.shape
    k_glob, n = w_mat.shape
    m_blk = m_glob // N_DEV
    n_half = n // N_SPLIT

    def body(x_ref, w_ref, out_ref, recv_buf, wbuf, amax_buf,
             send_sems, recv_sems, amax_send_sems, amax_recv_sems, wcp_sems):
        me = lax.axis_index("i")

        def origin(o):
            return lax.rem(me - o + N_DEV, N_DEV)

        sends = []
        for o in range(1, N_DEV):
            dst = lax.rem(me + o, N_DEV)
            snd = pltpu.make_async_remote_copy(
                src_ref=x_ref.at[pl.ds(dst * m_blk, m_blk), :],
                dst_ref=recv_buf.at[me],
                send_sem=send_sems.at[o - 1],
                recv_sem=recv_sems.at[me],
                device_id=(dst,),
                device_id_type=pl.DeviceIdType.MESH,
            )
            snd.start()
            sends.append(snd)

        recv_buf[me] = x_ref[pl.ds(me * m_blk, m_blk), :]

        def w_copy(o, h, slot):
            cp = pltpu.make_async_copy(
                w_ref.at[pl.ds(origin(o) * k_shard, k_shard),
                         pl.ds(h * n_half, n_half)],
                wbuf.at[slot],
                wcp_sems.at[slot],
            )
            cp.start()
            return cp

        chunks = [(o, h) for o in range(N_DEV) for h in range(N_SPLIT)]
        cps = {0: w_copy(0, 0, 0)}
        for c, (o, h) in enumerate(chunks):
            if c + 1 < len(chunks):
                o2, h2 = chunks[c + 1]
                cps[c + 1] = w_copy(o2, h2, (c + 1) % 2)
            if o > 0 and h == 0:
                src = origin(o)
                rcv = pltpu.make_async_remote_copy(
                    src_ref=x_ref.at[pl.ds(0, m_blk), :],
                    dst_ref=recv_buf.at[src],
                    send_sem=send_sems.at[0],
                    recv_sem=recv_sems.at[src],
                    device_id=(me,),
                    device_id_type=pl.DeviceIdType.MESH,
                )
                rcv.wait_recv()
            cps[c].wait()
            wb = wbuf[c % 2].astype(jnp.bfloat16)
            part = jnp.dot(recv_buf[origin(o)], wb,
                           preferred_element_type=jnp.float32)
            nsl = pl.ds(h * n_half, n_half)
            if o == 0:
                out_ref[:, nsl] = part
            else:
                out_ref[:, nsl] = out_ref[:, nsl] + part

        out_ref[:, :] = jnp.maximum(out_ref[:, :], 0.0)
        amax_buf[pl.ds(me, 1), :] = jnp.full(
            (1, 128), jnp.max(out_ref[:, :]), jnp.float32)

        a_sends = []
        for o in range(1, N_DEV):
            dst = lax.rem(me + o, N_DEV)
            asnd = pltpu.make_async_remote_copy(
                src_ref=amax_buf.at[pl.ds(me, 1), :],
                dst_ref=amax_buf.at[pl.ds(me, 1), :],
                send_sem=amax_send_sems.at[o - 1],
                recv_sem=amax_recv_sems.at[me],
                device_id=(dst,),
                device_id_type=pl.DeviceIdType.MESH,
            )
            asnd.start()
            a_sends.append(asnd)
        for o in range(1, N_DEV):
            src = origin(o)
            arcv = pltpu.make_async_remote_copy(
                src_ref=amax_buf.at[pl.ds(me, 1), :],
                dst_ref=amax_buf.at[pl.ds(src, 1), :],
                send_sem=amax_send_sems.at[0],
                recv_sem=amax_recv_sems.at[src],
                device_id=(me,),
                device_id_type=pl.DeviceIdType.MESH,
            )
            arcv.wait_recv()

        g_amax = jnp.max(amax_buf[:, :])
        scale = g_amax / 448.0
        for h in range(N_SPLIT):
            nsl = pl.ds(h * n_half, n_half)
            q = (out_ref[:, nsl] / scale).astype(jnp.float8_e4m3fn)
            out_ref[:, nsl] = q.astype(jnp.float32) * scale

        for s in sends:
            s.wait_send()
        for s in a_sends:
            s.wait_send()

    out = pl.pallas_call(
        body,
        out_shape=jax.ShapeDtypeStruct((m_blk, n), jnp.float32),
        in_specs=[
            pl.BlockSpec(memory_space=pltpu.MemorySpace.VMEM),
            pl.BlockSpec(memory_space=pl.ANY),
        ],
        out_specs=pl.BlockSpec(memory_space=pltpu.MemorySpace.VMEM),
        scratch_shapes=[
            pltpu.VMEM((N_DEV, m_blk, k_shard), jnp.bfloat16),
            pltpu.VMEM((2, k_shard, n_half), jnp.float32),
            pltpu.VMEM((N_DEV, 128), jnp.float32),
            pltpu.SemaphoreType.DMA((N_DEV - 1,)),
            pltpu.SemaphoreType.DMA((N_DEV,)),
            pltpu.SemaphoreType.DMA((N_DEV - 1,)),
            pltpu.SemaphoreType.DMA((N_DEV,)),
            pltpu.SemaphoreType.DMA((2,)),
        ],
    )(x.astype(jnp.bfloat16), w_mat)
    return out
